# baseline (speedup 1.0000x reference)
import jax
import jax.numpy as jnp
from jax import lax
from jax.experimental import pallas as pl
from jax.experimental.pallas import tpu as pltpu
from jax.experimental.pallas import tpu_sc as plsc

_E = 16
_K = 8
_H = 2048
_I = 768
_T = 512

_IB = 768
_NI = _I // _IB


def _moe_tc_body(w_ref, x_ref, gate_ref, up_ref, down_ref, out_ref):
    e = pl.program_id(0)
    i = pl.program_id(1)

    @pl.when((e == 0) & (i == 0))
    def _init():
        out_ref[...] = jnp.zeros_like(out_ref)

    x = x_ref[...].astype(jnp.bfloat16)
    g = lax.dot_general(x, gate_ref[0].astype(jnp.bfloat16),
                        (((1,), (1,)), ((), ())),
                        preferred_element_type=jnp.float32)
    u = lax.dot_general(x, up_ref[0].astype(jnp.bfloat16),
                        (((1,), (1,)), ((), ())),
                        preferred_element_type=jnp.float32)
    f = (g * jax.nn.sigmoid(g)) * u
    w = w_ref[0, 0, :]
    fw = (f * w[:, None]).astype(jnp.bfloat16)
    out_ref[...] += lax.dot_general(
        fw, down_ref[0].astype(jnp.bfloat16), (((1,), (1,)), ((), ())),
        preferred_element_type=jnp.float32)


def _moe_tc(w_et, hidden, gate_proj, up_proj, down_proj):
    return pl.pallas_call(
        _moe_tc_body,
        grid=(_E, _NI),
        in_specs=[
            pl.BlockSpec((1, 1, _T), lambda e, i: (e, 0, 0)),
            pl.BlockSpec((_T, _H), lambda e, i: (0, 0)),
            pl.BlockSpec((1, _IB, _H), lambda e, i: (e, i, 0)),
            pl.BlockSpec((1, _IB, _H), lambda e, i: (e, i, 0)),
            pl.BlockSpec((1, _H, _IB), lambda e, i: (e, 0, i)),
        ],
        out_specs=pl.BlockSpec((_T, _H), lambda e, i: (0, 0)),
        out_shape=jax.ShapeDtypeStruct((_T, _H), jnp.float32),
    )(w_et, hidden, gate_proj, up_proj, down_proj)


_NC = 2
_NS = 16
_TPW = _T // (_NC * _NS)


def _route_sc_body(logits_hbm, w_hbm, logits_v, w_v):
    wid = lax.axis_index("s") * _NC + lax.axis_index("c")
    base = wid * _TPW
    pltpu.sync_copy(logits_hbm.at[pl.ds(base, _TPW)], logits_v)
    tok = lax.iota(jnp.int32, _TPW)
    l = [plsc.load_gather(logits_v, [tok, jnp.full((_TPW,), e, jnp.int32)])
         for e in range(_E)]
    m = l[0]
    for e in range(1, _E):
        m = jnp.maximum(m, l[e])
    el = [jnp.exp(v - m) for v in l]
    s = el[0]
    for e in range(1, _E):
        s = s + el[e]
    p = [v / s for v in el]
    rank = [jnp.zeros((_TPW,), jnp.int32) for _ in range(_E)]
    one = jnp.ones((_TPW,), jnp.int32)
    for e1 in range(_E):
        for e2 in range(e1 + 1, _E):
            c = (p[e1] >= p[e2]).astype(jnp.int32)
            rank[e2] = rank[e2] + c
            rank[e1] = rank[e1] + (one - c)
    wsel = [jnp.where(rank[e] < _K, p[e], 0.0) for e in range(_E)]
    ns = wsel[0]
    for e in range(1, _E):
        ns = ns + wsel[e]
    for e in range(_E):
        w_v[e] = wsel[e] / ns
    pltpu.sync_copy(w_v, w_hbm.at[:, pl.ds(base, _TPW)])


@jax.jit
def _route_sc(router_logits):
    return pl.kernel(
        _route_sc_body,
        mesh=plsc.VectorSubcoreMesh(core_axis_name="c", subcore_axis_name="s"),
        compiler_params=pltpu.CompilerParams(
            needs_layout_passes=False, use_tc_tiling_on_sc=False),
        out_type=jax.ShapeDtypeStruct((_E, _T), jnp.float32),
        scratch_types=[
            pltpu.VMEM((_TPW, _E), jnp.float32),
            pltpu.VMEM((_E, _TPW), jnp.float32),
        ],
    )(router_logits)


def kernel(hidden_states, router_logits, gate_proj, up_proj, down_proj):
    w_et = _route_sc(router_logits.astype(jnp.float32))
    return _moe_tc(w_et.reshape(_E, 1, _T),
                   hidden_states, gate_proj, up_proj, down_proj)


# speedup vs baseline: 1.0012x; 1.0012x over previous
import jax
import jax.numpy as jnp
from jax import lax
from jax.experimental import pallas as pl
from jax.experimental.pallas import tpu as pltpu
from jax.experimental.pallas import tpu_sc as plsc

_E = 16
_K = 8
_H = 2048
_I = 768
_T = 512

_IB = 768
_NI = _I // _IB


def _moe_tc_body(w_ref, x_ref, gate_ref, up_ref, down_ref, out_ref):
    e = pl.program_id(0)
    i = pl.program_id(1)

    @pl.when((e == 0) & (i == 0))
    def _init():
        out_ref[...] = jnp.zeros_like(out_ref)

    x = x_ref[...]
    g = lax.dot_general(x, gate_ref[0], (((1,), (1,)), ((), ())),
                        preferred_element_type=jnp.float32)
    u = lax.dot_general(x, up_ref[0], (((1,), (1,)), ((), ())),
                        preferred_element_type=jnp.float32)
    f = (g * jax.nn.sigmoid(g)) * u
    w = w_ref[0, 0, :]
    fw = f * w[:, None]
    out_ref[...] += lax.dot_general(
        fw, down_ref[0], (((1,), (1,)), ((), ())),
        preferred_element_type=jnp.float32)


def _moe_tc(w_et, hidden, gate_proj, up_proj, down_proj):
    return pl.pallas_call(
        _moe_tc_body,
        grid=(_E, _NI),
        in_specs=[
            pl.BlockSpec((1, 1, _T), lambda e, i: (e, 0, 0)),
            pl.BlockSpec((_T, _H), lambda e, i: (0, 0)),
            pl.BlockSpec((1, _IB, _H), lambda e, i: (e, i, 0)),
            pl.BlockSpec((1, _IB, _H), lambda e, i: (e, i, 0)),
            pl.BlockSpec((1, _H, _IB), lambda e, i: (e, 0, i)),
        ],
        out_specs=pl.BlockSpec((_T, _H), lambda e, i: (0, 0)),
        out_shape=jax.ShapeDtypeStruct((_T, _H), jnp.float32),
    )(w_et, hidden, gate_proj, up_proj, down_proj)


_NC = 2
_NS = 16
_TPW = _T // (_NC * _NS)


def _route_sc_body(logits_hbm, w_hbm, logits_v, w_v):
    wid = lax.axis_index("s") * _NC + lax.axis_index("c")
    base = wid * _TPW
    pltpu.sync_copy(logits_hbm.at[pl.ds(base, _TPW)], logits_v)
    tok = lax.iota(jnp.int32, _TPW)
    l = [plsc.load_gather(logits_v, [tok, jnp.full((_TPW,), e, jnp.int32)])
         for e in range(_E)]
    m = l[0]
    for e in range(1, _E):
        m = jnp.maximum(m, l[e])
    el = [jnp.exp(v - m) for v in l]
    s = el[0]
    for e in range(1, _E):
        s = s + el[e]
    p = [v / s for v in el]
    rank = [jnp.zeros((_TPW,), jnp.int32) for _ in range(_E)]
    one = jnp.ones((_TPW,), jnp.int32)
    for e1 in range(_E):
        for e2 in range(e1 + 1, _E):
            c = (p[e1] >= p[e2]).astype(jnp.int32)
            rank[e2] = rank[e2] + c
            rank[e1] = rank[e1] + (one - c)
    wsel = [jnp.where(rank[e] < _K, p[e], 0.0) for e in range(_E)]
    ns = wsel[0]
    for e in range(1, _E):
        ns = ns + wsel[e]
    for e in range(_E):
        w_v[e] = wsel[e] / ns
    pltpu.sync_copy(w_v, w_hbm.at[:, pl.ds(base, _TPW)])


@jax.jit
def _route_sc(router_logits):
    return pl.kernel(
        _route_sc_body,
        mesh=plsc.VectorSubcoreMesh(core_axis_name="c", subcore_axis_name="s"),
        compiler_params=pltpu.CompilerParams(
            needs_layout_passes=False, use_tc_tiling_on_sc=False),
        out_type=jax.ShapeDtypeStruct((_E, _T), jnp.float32),
        scratch_types=[
            pltpu.VMEM((_TPW, _E), jnp.float32),
            pltpu.VMEM((_E, _TPW), jnp.float32),
        ],
    )(router_logits)


def kernel(hidden_states, router_logits, gate_proj, up_proj, down_proj):
    w_et = _route_sc(router_logits.astype(jnp.float32))
    return _moe_tc(w_et.reshape(_E, 1, _T),
                   hidden_states, gate_proj, up_proj, down_proj)
